# baseline (speedup 1.0000x reference)
import functools

import jax
import jax.numpy as jnp
from jax import lax
from jax.experimental import pallas as pl
from jax.experimental.pallas import tpu as pltpu
from jax.experimental.pallas import tpu_sc as plsc

_NC = 2
_NS = 16
_LANES = 16
_NW = _NC * _NS


def _main_body(adj_ref, x_ref, wenc_ref, wfea_ref, bfea_ref,
               wst_ref, bst_ref, fea_ref, st_ref, h_ref):
    @pl.when(pl.program_id(0) == 0)
    def _compute_h():
        h_ref[...] = jnp.dot(x_ref[...], wenc_ref[...],
                             preferred_element_type=jnp.float32)

    ne = jnp.maximum(jnp.dot(adj_ref[...], h_ref[...],
                             preferred_element_type=jnp.float32), 0.0)
    fea_ref[...] = lax.dot_general(
        ne, wfea_ref[...], (((1,), (1,)), ((), ())),
        preferred_element_type=jnp.float32) + bfea_ref[...]
    st_ref[...] = jnp.dot(ne, wst_ref[...],
                          preferred_element_type=jnp.float32) + bst_ref[...]


def _node_pass(x, adj, W_enc, W_fea, b_fea, W_st, b_st):
    n, d_feat = x.shape
    hid = W_enc.shape[1]
    rb = 400
    fea, st = pl.pallas_call(
        _main_body,
        grid=(pl.cdiv(n, rb),),
        in_specs=[
            pl.BlockSpec((rb, n), lambda i: (i, 0)),
            pl.BlockSpec((n, d_feat), lambda i: (0, 0)),
            pl.BlockSpec((d_feat, hid), lambda i: (0, 0)),
            pl.BlockSpec((hid, hid), lambda i: (0, 0)),
            pl.BlockSpec((1, hid), lambda i: (0, 0)),
            pl.BlockSpec((hid, 2), lambda i: (0, 0)),
            pl.BlockSpec((1, 2), lambda i: (0, 0)),
        ],
        out_specs=[
            pl.BlockSpec((rb, hid), lambda i: (i, 0)),
            pl.BlockSpec((rb, 2), lambda i: (i, 0)),
        ],
        out_shape=[
            jax.ShapeDtypeStruct((n, hid), jnp.float32),
            jax.ShapeDtypeStruct((n, 2), jnp.float32),
        ],
        scratch_shapes=[pltpu.VMEM((n, hid), jnp.float32)],
        compiler_params=pltpu.CompilerParams(
            dimension_semantics=("arbitrary",)),
    )(adj, x, W_enc, W_fea, b_fea.reshape(1, hid), W_st, b_st)
    return fea, st


def _edge_pass(st_flat, src, dst):
    n2 = st_flat.shape[0]
    e = src.shape[0]
    ch = e // _NW
    unroll = 25
    assert ch % (_LANES * unroll) == 0

    mesh = plsc.VectorSubcoreMesh(core_axis_name="c", subcore_axis_name="s",
                                  num_cores=_NC, num_subcores=_NS)

    @functools.partial(
        pl.kernel,
        mesh=mesh,
        out_type=jax.ShapeDtypeStruct((e,), jnp.float32),
        compiler_params=pltpu.CompilerParams(needs_layout_passes=False),
        scratch_types=[
            pltpu.VMEM((n2,), jnp.float32),
            pltpu.VMEM((ch,), jnp.int32),
            pltpu.VMEM((ch,), jnp.int32),
            pltpu.VMEM((ch,), jnp.float32),
            pltpu.SemaphoreType.DMA,
            pltpu.SemaphoreType.DMA,
            pltpu.SemaphoreType.DMA,
        ],
    )
    def edge_kernel(st_hbm, src_hbm, dst_hbm, out_hbm,
                    st_v, src_v, dst_v, out_v, sem0, sem1, sem2):
        wid = lax.axis_index("s") * _NC + lax.axis_index("c")
        base = wid * ch
        c0 = pltpu.async_copy(st_hbm, st_v, sem0)
        c1 = pltpu.async_copy(src_hbm.at[pl.ds(base, ch)], src_v, sem1)
        c2 = pltpu.async_copy(dst_hbm.at[pl.ds(base, ch)], dst_v, sem2)
        c0.wait()
        c1.wait()
        c2.wait()

        @plsc.parallel_loop(0, ch, _LANES, unroll=unroll)
        def _gather_body(o):
            gs = plsc.load_gather(st_v, [src_v[pl.ds(o, _LANES)] * 2])
            gt = plsc.load_gather(st_v, [dst_v[pl.ds(o, _LANES)] * 2 + 1])
            out_v[pl.ds(o, _LANES)] = gs + gt
        pltpu.sync_copy(out_v, out_hbm.at[pl.ds(base, ch)])

    return edge_kernel(st_flat, src, dst)


def kernel(x, adj, edge_index, W_enc, W_edge, b_edge, W_fea, b_fea):
    hid = W_enc.shape[1]
    e = edge_index.shape[1]
    src = edge_index[0].astype(jnp.int32)
    dst = edge_index[1].astype(jnp.int32)
    W_st = W_edge.reshape(2, hid).T
    b_st = jnp.stack([b_edge[0], jnp.zeros((), jnp.float32)]).reshape(1, 2)

    fea_logits, st = _node_pass(x, adj, W_enc, W_fea, b_fea, W_st, b_st)
    edge_flat = _edge_pass(st.reshape(-1), src, dst)
    return (edge_flat.reshape(e, 1), fea_logits)


# speedup vs baseline: 1.0181x; 1.0181x over previous
import functools

import jax
import jax.numpy as jnp
from jax import lax
from jax.experimental import pallas as pl
from jax.experimental.pallas import tpu as pltpu
from jax.experimental.pallas import tpu_sc as plsc

_NC = 2
_NS = 16
_LANES = 16
_NW = _NC * _NS


def _main_body(adj_ref, x_ref, wenc_ref, wfea_ref, bfea_ref,
               wst_ref, bst_ref, fea_ref, st_ref, h_ref):
    @pl.when(pl.program_id(0) == 0)
    def _compute_h():
        h_ref[...] = jnp.dot(x_ref[...], wenc_ref[...],
                             preferred_element_type=jnp.float32)

    ne = jnp.maximum(jnp.dot(adj_ref[...], h_ref[...],
                             preferred_element_type=jnp.float32), 0.0)
    fea_ref[...] = lax.dot_general(
        ne, wfea_ref[...], (((1,), (1,)), ((), ())),
        preferred_element_type=jnp.float32) + bfea_ref[...]
    st_ref[...] = jnp.dot(ne, wst_ref[...],
                          preferred_element_type=jnp.float32) + bst_ref[...]


def _node_pass(x, adj, W_enc, W_fea, b_fea, W_st, b_st):
    n, d_feat = x.shape
    hid = W_enc.shape[1]
    rb = 400
    fea, st = pl.pallas_call(
        _main_body,
        grid=(pl.cdiv(n, rb),),
        in_specs=[
            pl.BlockSpec((rb, n), lambda i: (i, 0)),
            pl.BlockSpec((n, d_feat), lambda i: (0, 0)),
            pl.BlockSpec((d_feat, hid), lambda i: (0, 0)),
            pl.BlockSpec((hid, hid), lambda i: (0, 0)),
            pl.BlockSpec((1, hid), lambda i: (0, 0)),
            pl.BlockSpec((hid, 2), lambda i: (0, 0)),
            pl.BlockSpec((1, 2), lambda i: (0, 0)),
        ],
        out_specs=[
            pl.BlockSpec((rb, hid), lambda i: (i, 0)),
            pl.BlockSpec((rb, 2), lambda i: (i, 0)),
        ],
        out_shape=[
            jax.ShapeDtypeStruct((n, hid), jnp.float32),
            jax.ShapeDtypeStruct((n, 2), jnp.float32),
        ],
        scratch_shapes=[pltpu.VMEM((n, hid), jnp.float32)],
        compiler_params=pltpu.CompilerParams(
            dimension_semantics=("arbitrary",)),
    )(adj, x, W_enc, W_fea, b_fea.reshape(1, hid), W_st, b_st)
    return fea, st


def _edge_pass(st_flat, src, dst):
    n2 = st_flat.shape[0]
    e = src.shape[0]
    ch = e // _NW
    unroll = 5
    assert ch % (_LANES * unroll) == 0

    mesh = plsc.VectorSubcoreMesh(core_axis_name="c", subcore_axis_name="s",
                                  num_cores=_NC, num_subcores=_NS)

    @functools.partial(
        pl.kernel,
        mesh=mesh,
        out_type=jax.ShapeDtypeStruct((e,), jnp.float32),
        compiler_params=pltpu.CompilerParams(needs_layout_passes=False),
        scratch_types=[
            pltpu.VMEM((n2,), jnp.float32),
            pltpu.VMEM((ch,), jnp.int32),
            pltpu.VMEM((ch,), jnp.int32),
            pltpu.VMEM((ch,), jnp.float32),
            pltpu.SemaphoreType.DMA,
            pltpu.SemaphoreType.DMA,
            pltpu.SemaphoreType.DMA,
        ],
    )
    def edge_kernel(st_hbm, src_hbm, dst_hbm, out_hbm,
                    st_v, src_v, dst_v, out_v, sem0, sem1, sem2):
        wid = lax.axis_index("s") * _NC + lax.axis_index("c")
        base = wid * ch
        c0 = pltpu.async_copy(st_hbm, st_v, sem0)
        c1 = pltpu.async_copy(src_hbm.at[pl.ds(base, ch)], src_v, sem1)
        c2 = pltpu.async_copy(dst_hbm.at[pl.ds(base, ch)], dst_v, sem2)
        c0.wait()
        c1.wait()
        c2.wait()

        @plsc.parallel_loop(0, ch, _LANES, unroll=unroll)
        def _gather_body(o):
            gs = plsc.load_gather(st_v, [src_v[pl.ds(o, _LANES)] * 2])
            gt = plsc.load_gather(st_v, [dst_v[pl.ds(o, _LANES)] * 2 + 1])
            out_v[pl.ds(o, _LANES)] = gs + gt
        pltpu.sync_copy(out_v, out_hbm.at[pl.ds(base, ch)])

    return edge_kernel(st_flat, src, dst)


def kernel(x, adj, edge_index, W_enc, W_edge, b_edge, W_fea, b_fea):
    hid = W_enc.shape[1]
    e = edge_index.shape[1]
    src = edge_index[0].astype(jnp.int32)
    dst = edge_index[1].astype(jnp.int32)
    W_st = W_edge.reshape(2, hid).T
    b_st = jnp.stack([b_edge[0], jnp.zeros((), jnp.float32)]).reshape(1, 2)

    fea_logits, st = _node_pass(x, adj, W_enc, W_fea, b_fea, W_st, b_st)
    edge_flat = _edge_pass(st.reshape(-1), src, dst)
    return (edge_flat.reshape(e, 1), fea_logits)
